# baseline (speedup 1.0000x reference)
import functools

import jax
import jax.numpy as jnp
from jax import lax
from jax.experimental import pallas as pl
from jax.experimental.pallas import tpu as pltpu
from jax.experimental.pallas import tpu_sc as plsc

N_NODES = 10000
D_FEAT = 128
N_EDGES = 320000

NUM_CORES = 2
NUM_SUBCORES = 16
NUM_WORKERS = NUM_CORES * NUM_SUBCORES
CHUNK = 32
CPW = 320
EDGES_PER_WORKER = CPW * CHUNK
E_PAD = NUM_WORKERS * EDGES_PER_WORKER
N_PAD = 10000
ROWS_PER_TILE = N_PAD // NUM_SUBCORES
IDXSUB = 40


@functools.partial(
    pl.kernel,
    out_type=jax.ShapeDtypeStruct((NUM_CORES * N_PAD, D_FEAT), jnp.float32),
    mesh=plsc.VectorSubcoreMesh(core_axis_name="c", subcore_axis_name="s"),
    compiler_params=pltpu.CompilerParams(use_tc_tiling_on_sc=False),
    scratch_types=[
        pltpu.VMEM_SHARED((N_PAD, D_FEAT), jnp.float32),
        pltpu.VMEM((CPW, CHUNK), jnp.int32),
        pltpu.VMEM((CPW, CHUNK), jnp.int32),
        pltpu.VMEM((EDGES_PER_WORKER,), jnp.float32),
        pltpu.VMEM((CHUNK, D_FEAT), jnp.float32),
        pltpu.VMEM((CHUNK, D_FEAT), jnp.float32),
        pltpu.VMEM((CHUNK, D_FEAT), jnp.float32),
        pltpu.VMEM((CHUNK, D_FEAT), jnp.float32),
        pltpu.SemaphoreType.DMA,
        pltpu.SemaphoreType.DMA,
        pltpu.SemaphoreType.DMA,
        pltpu.SemaphoreType.DMA,
    ],
)
def _conv_pass(x_hbm, src_hbm, dst_hbm, w_hbm, out_hbm,
               acc, src_v, dst_v, w_v, g0, g1, s0, s1,
               gsem0, gsem1, ssem0, ssem1):
    c = lax.axis_index("c")
    s = lax.axis_index("s")
    wid = c * NUM_SUBCORES + s
    gbuf = (g0, g1)
    sbuf = (s0, s1)
    gsem = (gsem0, gsem1)
    ssem = (ssem0, ssem1)

    ebase = wid * EDGES_PER_WORKER
    cbase = wid * CPW
    WSUB = EDGES_PER_WORKER // 4

    def load_idx(k, _):
        sl_h = pl.ds(cbase + k * IDXSUB, IDXSUB)
        sl_v = pl.ds(k * IDXSUB, IDXSUB)
        pltpu.sync_copy(src_hbm.at[sl_h], src_v.at[sl_v])
        pltpu.sync_copy(dst_hbm.at[sl_h], dst_v.at[sl_v])
        return 0

    lax.fori_loop(0, CPW // IDXSUB, load_idx, 0)

    def load_w(k, _):
        pltpu.sync_copy(w_hbm.at[pl.ds(ebase + k * WSUB, WSUB)],
                        w_v.at[pl.ds(k * WSUB, WSUB)])
        return 0

    lax.fori_loop(0, 4, load_w, 0)

    def start_gather(ci, p):
        pltpu.async_copy(x_hbm.at[src_v.at[ci]], gbuf[p], gsem[p])

    def wait_gather(ci, p):
        pltpu.make_async_copy(x_hbm.at[src_v.at[ci]], gbuf[p], gsem[p]).wait()

    def start_scatter(ci, p):
        pltpu.async_copy(sbuf[p], acc.at[dst_v.at[ci]], ssem[p], add=True)

    def wait_scatter(ci, p):
        pltpu.make_async_copy(sbuf[p], acc.at[dst_v.at[ci]], ssem[p]).wait()

    start_gather(0, 0)
    start_gather(1, 1)

    zvec = jnp.zeros((16,), jnp.float32)

    def zero_rows(i, _):
        for j in range(D_FEAT // 16):
            s0[i, pl.ds(j * 16, 16)] = zvec
        return 0

    lax.fori_loop(0, CHUNK, zero_rows, 0)
    row0 = s * ROWS_PER_TILE
    for k in range(ROWS_PER_TILE // CHUNK):
        pltpu.sync_copy(s0, acc.at[pl.ds(row0 + k * CHUNK, CHUNK)])
    zrem = ROWS_PER_TILE % CHUNK
    if zrem:
        pltpu.sync_copy(s0.at[pl.ds(0, zrem)],
                        acc.at[pl.ds(row0 + (ROWS_PER_TILE // CHUNK) * CHUNK, zrem)])
    plsc.subcore_barrier()

    def scale(ci, p):
        g, sb = gbuf[p], sbuf[p]

        def scale_group(gi, _):
            wvec = w_v[pl.ds(ci * CHUNK + gi * 16, 16)]
            for l in range(16):
                e = gi * 16 + l
                wsp = wvec[l]
                for j in range(D_FEAT // 16):
                    sl = pl.ds(j * 16, 16)
                    sb[e, sl] = g[e, sl] * wsp
            return 0

        lax.fori_loop(0, CHUNK // 16, scale_group, 0)

    for ci in (0, 1):
        p = ci % 2
        wait_gather(ci, p)
        scale(ci, p)
        start_gather(ci + 2, p)
        start_scatter(ci, p)

    def pair_body(k, _):
        for j in range(2):
            ci = 2 * k + j
            wait_gather(ci, j)
            wait_scatter(ci - 2, j)
            scale(ci, j)
            start_gather(ci + 2, j)
            start_scatter(ci, j)
        return 0

    lax.fori_loop(1, CPW // 2 - 1, pair_body, 0)

    for ci in (CPW - 2, CPW - 1):
        p = ci % 2
        wait_gather(ci, p)
        wait_scatter(ci - 2, p)
        scale(ci, p)
        start_scatter(ci, p)

    wait_scatter(CPW - 2, 0)
    wait_scatter(CPW - 1, 1)
    plsc.subcore_barrier()

    out0 = c * N_PAD + row0
    pltpu.sync_copy(acc.at[pl.ds(row0, ROWS_PER_TILE)],
                    out_hbm.at[pl.ds(out0, ROWS_PER_TILE)])


def _add_body(a_ref, b_ref, o_ref):
    o_ref[...] = a_ref[...] + b_ref[...]


def _add_relu_body(a_ref, b_ref, o_ref):
    o_ref[...] = jnp.maximum(a_ref[...] + b_ref[...], 0.0)


def _combine(p0, p1, relu):
    body = _add_relu_body if relu else _add_body
    blk = 1000
    return pl.pallas_call(
        body,
        grid=(N_PAD // blk,),
        in_specs=[pl.BlockSpec((blk, D_FEAT), lambda i: (i, 0)),
                  pl.BlockSpec((blk, D_FEAT), lambda i: (i, 0))],
        out_specs=pl.BlockSpec((blk, D_FEAT), lambda i: (i, 0)),
        out_shape=jax.ShapeDtypeStruct((N_PAD, D_FEAT), jnp.float32),
    )(p0, p1)


def kernel(x, edge_index, edge_weight):
    src = edge_index[0].astype(jnp.int32)
    dst = edge_index[1].astype(jnp.int32)
    w = edge_weight.astype(jnp.float32)
    pad = E_PAD - N_EDGES
    zpad = jnp.zeros((pad,), jnp.int32)
    src = jnp.concatenate([src, zpad]).reshape(NUM_WORKERS * CPW, CHUNK)
    dst = jnp.concatenate([dst, zpad]).reshape(NUM_WORKERS * CPW, CHUNK)
    w = jnp.concatenate([w, jnp.zeros((pad,), jnp.float32)])

    p = _conv_pass(x, src, dst, w)
    h = _combine(p[:N_PAD], p[N_PAD:], relu=False)
    p2 = _conv_pass(h, src, dst, w)
    out = _combine(p2[:N_PAD], p2[N_PAD:], relu=True)
    return out[:N_NODES]


# speedup vs baseline: 2.5984x; 2.5984x over previous
import functools

import jax
import jax.numpy as jnp
from jax import lax
from jax.experimental import pallas as pl
from jax.experimental.pallas import tpu as pltpu
from jax.experimental.pallas import tpu_sc as plsc

N_NODES = 10000
D_FEAT = 128
N_EDGES = 320000

NUM_CORES = 2
NUM_SUBCORES = 16
NUM_WORKERS = NUM_CORES * NUM_SUBCORES
CHUNK = 32
CPW = 320
EDGES_PER_WORKER = CPW * CHUNK
E_PAD = NUM_WORKERS * EDGES_PER_WORKER
N_PAD = 10000
ROWS_PER_TILE = N_PAD // NUM_SUBCORES
IDXSUB = 40


@functools.partial(
    pl.kernel,
    out_type=jax.ShapeDtypeStruct((NUM_CORES * N_PAD, D_FEAT), jnp.float32),
    mesh=plsc.VectorSubcoreMesh(core_axis_name="c", subcore_axis_name="s"),
    compiler_params=pltpu.CompilerParams(use_tc_tiling_on_sc=False),
    scratch_types=[
        pltpu.VMEM_SHARED((N_PAD, D_FEAT), jnp.float32),
        pltpu.VMEM((CPW, CHUNK), jnp.int32),
        pltpu.VMEM((CPW, CHUNK), jnp.int32),
        pltpu.VMEM((EDGES_PER_WORKER,), jnp.float32),
        pltpu.VMEM((CHUNK, D_FEAT), jnp.float32),
        pltpu.VMEM((CHUNK, D_FEAT), jnp.float32),
        pltpu.VMEM((CHUNK, D_FEAT), jnp.float32),
        pltpu.VMEM((CHUNK, D_FEAT), jnp.float32),
        pltpu.SemaphoreType.DMA,
        pltpu.SemaphoreType.DMA,
        pltpu.SemaphoreType.DMA,
        pltpu.SemaphoreType.DMA,
    ],
)
def _conv_pass(x_hbm, src_hbm, dst_hbm, w_hbm, out_hbm,
               acc, src_v, dst_v, w_v, g0, g1, s0, s1,
               gsem0, gsem1, ssem0, ssem1):
    c = lax.axis_index("c")
    s = lax.axis_index("s")
    wid = c * NUM_SUBCORES + s
    gbuf = (g0, g1)
    sbuf = (s0, s1)
    gsem = (gsem0, gsem1)
    ssem = (ssem0, ssem1)

    ebase = wid * EDGES_PER_WORKER
    cbase = wid * CPW
    WSUB = EDGES_PER_WORKER // 4

    def load_idx(k, _):
        sl_h = pl.ds(cbase + k * IDXSUB, IDXSUB)
        sl_v = pl.ds(k * IDXSUB, IDXSUB)
        pltpu.sync_copy(src_hbm.at[sl_h], src_v.at[sl_v])
        pltpu.sync_copy(dst_hbm.at[sl_h], dst_v.at[sl_v])
        return 0

    lax.fori_loop(0, CPW // IDXSUB, load_idx, 0)

    def load_w(k, _):
        pltpu.sync_copy(w_hbm.at[pl.ds(ebase + k * WSUB, WSUB)],
                        w_v.at[pl.ds(k * WSUB, WSUB)])
        return 0

    lax.fori_loop(0, 4, load_w, 0)

    def start_gather(ci, p):
        pltpu.async_copy(x_hbm.at[src_v.at[ci]], gbuf[p], gsem[p])

    def wait_gather(ci, p):
        pltpu.make_async_copy(x_hbm.at[src_v.at[ci]], gbuf[p], gsem[p]).wait()

    def start_scatter(ci, p):
        pltpu.async_copy(sbuf[p], acc.at[dst_v.at[ci]], ssem[p], add=True)

    def wait_scatter(ci, p):
        pltpu.make_async_copy(sbuf[p], acc.at[dst_v.at[ci]], ssem[p]).wait()

    start_gather(0, 0)
    start_gather(1, 1)

    zvec = jnp.zeros((16,), jnp.float32)

    def zero_rows(i, _):
        for j in range(D_FEAT // 16):
            s0[i, pl.ds(j * 16, 16)] = zvec
        return 0

    lax.fori_loop(0, CHUNK, zero_rows, 0)
    row0 = s * ROWS_PER_TILE
    for k in range(ROWS_PER_TILE // CHUNK):
        pltpu.sync_copy(s0, acc.at[pl.ds(row0 + k * CHUNK, CHUNK)])
    zrem = ROWS_PER_TILE % CHUNK
    if zrem:
        pltpu.sync_copy(s0.at[pl.ds(0, zrem)],
                        acc.at[pl.ds(row0 + (ROWS_PER_TILE // CHUNK) * CHUNK, zrem)])
    plsc.subcore_barrier()

    def scale(ci, p):
        g, sb = gbuf[p], sbuf[p]

        def scale_group(gi, _):
            wvec = w_v[pl.ds(ci * CHUNK + gi * 16, 16)]
            for l in range(16):
                e = gi * 16 + l
                wsp = wvec[l]
                for j in range(D_FEAT // 16):
                    sl = pl.ds(j * 16, 16)
                    sb[e, sl] = g[e, sl] * wsp
            return 0

        lax.fori_loop(0, CHUNK // 16, scale_group, 0)

    for ci in (0, 1):
        p = ci % 2
        wait_gather(ci, p)
        scale(ci, p)
        start_gather(ci + 2, p)
        start_scatter(ci, p)

    def pair_body(k, _):
        for j in range(2):
            ci = 2 * k + j
            wait_gather(ci, j)
            wait_scatter(ci - 2, j)
            scale(ci, j)
            start_gather(ci + 2, j)
            start_scatter(ci, j)
        return 0

    lax.fori_loop(1, CPW // 2 - 1, pair_body, 0)

    for ci in (CPW - 2, CPW - 1):
        p = ci % 2
        wait_gather(ci, p)
        wait_scatter(ci - 2, p)
        scale(ci, p)
        start_scatter(ci, p)

    wait_scatter(CPW - 2, 0)
    wait_scatter(CPW - 1, 1)
    plsc.subcore_barrier()

    out0 = c * N_PAD + row0
    pltpu.sync_copy(acc.at[pl.ds(row0, ROWS_PER_TILE)],
                    out_hbm.at[pl.ds(out0, ROWS_PER_TILE)])


def _add_body(a_ref, b_ref, o_ref):
    o_ref[...] = a_ref[...] + b_ref[...]


def _add_relu_body(a_ref, b_ref, o_ref):
    o_ref[...] = jnp.maximum(a_ref[...] + b_ref[...], 0.0)


def _combine(p0, p1, relu):
    body = _add_relu_body if relu else _add_body
    blk = 1000
    return pl.pallas_call(
        body,
        grid=(N_PAD // blk,),
        in_specs=[pl.BlockSpec((blk, D_FEAT), lambda i: (i, 0)),
                  pl.BlockSpec((blk, D_FEAT), lambda i: (i, 0))],
        out_specs=pl.BlockSpec((blk, D_FEAT), lambda i: (i, 0)),
        out_shape=jax.ShapeDtypeStruct((N_PAD, D_FEAT), jnp.float32),
    )(p0, p1)


def kernel(x, edge_index, edge_weight):
    src = edge_index[0].astype(jnp.int32)
    dst = edge_index[1].astype(jnp.int32)
    w = edge_weight.astype(jnp.float32)
    pad = E_PAD - N_EDGES
    spread = jnp.arange(pad, dtype=jnp.int32) % N_NODES
    src = jnp.concatenate([src, spread]).reshape(NUM_WORKERS * CPW, CHUNK)
    dst = jnp.concatenate([dst, spread]).reshape(NUM_WORKERS * CPW, CHUNK)
    w = jnp.concatenate([w, jnp.zeros((pad,), jnp.float32)])

    p = _conv_pass(x, src, dst, w)
    h = _combine(p[:N_PAD], p[N_PAD:], relu=False)
    p2 = _conv_pass(h, src, dst, w)
    out = _combine(p2[:N_PAD], p2[N_PAD:], relu=True)
    return out[:N_NODES]
